# baseline (speedup 1.0000x reference)
import functools

import jax
import jax.numpy as jnp
from jax import lax
from jax.experimental import pallas as pl
from jax.experimental.pallas import tpu as pltpu
from jax.experimental.pallas import tpu_sc as plsc

B = 16384
F = 32
NC = 2
NS = 16
NW = NC * NS
L = 16
TS = 8
TL = 128
HALF = F // 2
PROWS = L * HALF

B_SC = 10240
B_TC = B - B_SC
BPW = B_SC // NW
UPT = 16


def _sc_body(users_hbm, items_hbm, utab_hbm, itab_hbm, params_hbm, out_hbm,
             idx_u, idx_i, pan_u, pan_i, params_v, out_v, sem_u, sem_i):
    wid = lax.axis_index("s") * NC + lax.axis_index("c")
    base = wid * BPW

    pltpu.sync_copy(users_hbm.at[pl.ds(base, BPW)], idx_u)
    pltpu.sync_copy(items_hbm.at[pl.ds(base, BPW)], idx_i)
    pltpu.sync_copy(params_hbm, params_v)

    wv = [params_v[pl.ds(k * L, L)] for k in range(F // L)]
    bv = params_v[pl.ds(F // L * L, L)]
    w = [wv[f // L][f % L] for f in range(F)]
    bias = bv[0]
    lane = lax.iota(jnp.int32, L)

    def wave(v, carry):
        uvec = idx_u[pl.ds(v * L, L)]
        ivec = idx_i[pl.ds(v * L, L)]
        rem_u = uvec - (uvec // TL) * TL
        rem_i = ivec - (ivec // TL) * TL
        acc = jnp.zeros((L,), jnp.float32)
        for half in range(2):
            fr = half * HALF
            copies = []
            for k in range(L):
                qu = pl.multiple_of((uvec[k] // TL) * TL, TL)
                qi = pl.multiple_of((ivec[k] // TL) * TL, TL)
                copies.append(pltpu.async_copy(
                    utab_hbm.at[pl.ds(fr, HALF), pl.ds(qu, TL)],
                    pan_u.at[pl.ds(k * HALF, HALF)], sem_u))
                copies.append(pltpu.async_copy(
                    itab_hbm.at[pl.ds(fr, HALF), pl.ds(qi, TL)],
                    pan_i.at[pl.ds(k * HALF, HALF)], sem_i))
            for c in copies:
                c.wait()
            for fo in range(HALF):
                f = half * HALF + fo
                rows = lane * HALF + fo
                ucol = plsc.load_gather(pan_u, [rows, rem_u])
                icol = plsc.load_gather(pan_i, [rows, rem_i])
                acc = acc + (ucol * icol) * w[f]
        z = acc + bias
        out_v[pl.ds(v * L, L)] = 1.0 / (1.0 + jnp.exp(-z))
        return carry

    lax.fori_loop(0, BPW // L, wave, 0)

    pltpu.sync_copy(out_v, out_hbm.at[pl.ds(base, BPW)])


_gmf_sc = functools.partial(
    pl.kernel,
    out_type=jax.ShapeDtypeStruct((B_SC,), jnp.float32),
    mesh=plsc.VectorSubcoreMesh(core_axis_name="c", subcore_axis_name="s"),
    scratch_types=[
        pltpu.VMEM((BPW,), jnp.int32),
        pltpu.VMEM((BPW,), jnp.int32),
        pltpu.VMEM((PROWS, TL), jnp.float32),
        pltpu.VMEM((PROWS, TL), jnp.float32),
        pltpu.VMEM((48,), jnp.float32),
        pltpu.VMEM((BPW,), jnp.float32),
        pltpu.SemaphoreType.DMA,
        pltpu.SemaphoreType.DMA,
    ],
    compiler_params=pltpu.CompilerParams(needs_layout_passes=False),
)(_sc_body)


def _tc_body(uref, iref, *refs):
    upans = refs[:UPT]
    ipans = refs[UPT:2 * UPT]
    params_ref = refs[2 * UPT]
    out_ref = refs[2 * UPT + 1]
    j = pl.program_id(0)
    w = params_ref[0, :F]
    bias = params_ref[0, F]
    col_iota = lax.broadcasted_iota(jnp.int32, (F, TL), 1)
    zs = []
    for k in range(UPT):
        cu = uref[j * UPT + k] % TL
        ci = iref[j * UPT + k] % TL
        pu = upans[k][...]
        pi = ipans[k][...]
        su = jnp.where(col_iota == cu, pu, 0.0).sum(axis=1)
        si = jnp.where(col_iota == ci, pi, 0.0).sum(axis=1)
        zs.append(jnp.sum(su * si * w) + bias)
    z = jnp.stack(zs)
    out_ref[0, 0, :] = 1.0 / (1.0 + jnp.exp(-z))


def _tc_call(users, items, utab_t, itab_t, params2d):
    grid = (B_TC // UPT,)
    upan_specs = [
        pl.BlockSpec((F, TL), functools.partial(
            lambda j, uref, iref, kk: (0, uref[j * UPT + kk] // TL), kk=k))
        for k in range(UPT)
    ]
    ipan_specs = [
        pl.BlockSpec((F, TL), functools.partial(
            lambda j, uref, iref, kk: (0, iref[j * UPT + kk] // TL), kk=k))
        for k in range(UPT)
    ]
    param_spec = pl.BlockSpec((1, 48), lambda j, uref, iref: (0, 0))
    grid_spec = pltpu.PrefetchScalarGridSpec(
        num_scalar_prefetch=2,
        grid=grid,
        in_specs=[*upan_specs, *ipan_specs, param_spec],
        out_specs=pl.BlockSpec((1, 1, UPT), lambda j, uref, iref: (j, 0, 0)),
    )
    out = pl.pallas_call(
        _tc_body,
        grid_spec=grid_spec,
        out_shape=jax.ShapeDtypeStruct((B_TC // UPT, 1, UPT), jnp.float32),
    )(users, items, *([utab_t] * UPT), *([itab_t] * UPT), params2d)
    return out.reshape(-1)


def kernel(users, items, user_table, item_table, pred_w, pred_b):
    params = jnp.concatenate([
        pred_w.reshape(-1).astype(jnp.float32),
        pred_b.reshape(-1).astype(jnp.float32),
        jnp.zeros((48 - F - 1,), jnp.float32),
    ])
    users = users.astype(jnp.int32)
    items = items.astype(jnp.int32)
    utab_t = user_table.T
    itab_t = item_table.T
    out_sc = _gmf_sc(users[:B_SC], items[:B_SC], utab_t, itab_t, params)
    out_tc = _tc_call(users[B_SC:], items[B_SC:], utab_t, itab_t,
                      params.reshape(1, 48))
    return jnp.concatenate([out_sc, out_tc])


# speedup vs baseline: 4.1233x; 4.1233x over previous
import functools

import jax
import jax.numpy as jnp
from jax import lax
from jax.experimental import pallas as pl
from jax.experimental.pallas import tpu as pltpu
from jax.experimental.pallas import tpu_sc as plsc

B = 16384
F = 32
NC = 2
NS = 16
NW = NC * NS
BPW = B // NW
L = 16
TS = 8
TL = 128
HALF = F // 2
NWAVE = BPW // L
PROWS = L * HALF


def _gmf_body(users_hbm, items_hbm, utab_hbm, itab_hbm, params_hbm, out_hbm,
              idx_u, idx_i, pan_u, pan_i, params_v, out_v, sem_u, sem_i):
    wid = lax.axis_index("s") * NC + lax.axis_index("c")
    base = wid * BPW

    pltpu.sync_copy(users_hbm.at[pl.ds(base, BPW)], idx_u)
    pltpu.sync_copy(items_hbm.at[pl.ds(base, BPW)], idx_i)
    pltpu.sync_copy(params_hbm, params_v)

    wv = [params_v[pl.ds(k * L, L)] for k in range(F // L)]
    bv = params_v[pl.ds(F // L * L, L)]
    w = [wv[f // L][f % L] for f in range(F)]
    bias = bv[0]
    lane = lax.iota(jnp.int32, L)

    def wave(v, carry):
        uvec = idx_u[pl.ds(v * L, L)]
        ivec = idx_i[pl.ds(v * L, L)]
        rem_u = uvec - (uvec // TL) * TL
        rem_i = ivec - (ivec // TL) * TL
        acc = jnp.zeros((L,), jnp.float32)
        for half in range(2):
            fr = half * HALF
            copies = []
            for k in range(L):
                qu = pl.multiple_of((uvec[k] // TL) * TL, TL)
                qi = pl.multiple_of((ivec[k] // TL) * TL, TL)
                copies.append(pltpu.async_copy(
                    utab_hbm.at[pl.ds(fr, HALF), pl.ds(qu, TL)],
                    pan_u.at[pl.ds(k * HALF, HALF)], sem_u))
                copies.append(pltpu.async_copy(
                    itab_hbm.at[pl.ds(fr, HALF), pl.ds(qi, TL)],
                    pan_i.at[pl.ds(k * HALF, HALF)], sem_i))
            for c in copies:
                c.wait()
            for fo in range(HALF):
                f = half * HALF + fo
                rows = lane * HALF + fo
                ucol = plsc.load_gather(pan_u, [rows, rem_u])
                icol = plsc.load_gather(pan_i, [rows, rem_i])
                acc = acc + (ucol * icol) * w[f]
        z = acc + bias
        out_v[pl.ds(v * L, L)] = 1.0 / (1.0 + jnp.exp(-z))
        return carry

    lax.fori_loop(0, NWAVE, wave, 0)

    pltpu.sync_copy(out_v, out_hbm.at[pl.ds(base, BPW)])


_gmf = functools.partial(
    pl.kernel,
    out_type=jax.ShapeDtypeStruct((B,), jnp.float32),
    mesh=plsc.VectorSubcoreMesh(core_axis_name="c", subcore_axis_name="s"),
    scratch_types=[
        pltpu.VMEM((BPW,), jnp.int32),
        pltpu.VMEM((BPW,), jnp.int32),
        pltpu.VMEM((PROWS, TL), jnp.float32),
        pltpu.VMEM((PROWS, TL), jnp.float32),
        pltpu.VMEM((48,), jnp.float32),
        pltpu.VMEM((BPW,), jnp.float32),
        pltpu.SemaphoreType.DMA,
        pltpu.SemaphoreType.DMA,
    ],
    compiler_params=pltpu.CompilerParams(needs_layout_passes=False),
)(_gmf_body)


def kernel(users, items, user_table, item_table, pred_w, pred_b):
    params = jnp.concatenate([
        pred_w.reshape(-1).astype(jnp.float32),
        pred_b.reshape(-1).astype(jnp.float32),
        jnp.zeros((48 - F - 1,), jnp.float32),
    ])
    return _gmf(users.astype(jnp.int32), items.astype(jnp.int32),
                user_table.T, item_table.T, params)


# speedup vs baseline: 4.1845x; 1.0148x over previous
import functools

import jax
import jax.numpy as jnp
from jax import lax
from jax.experimental import pallas as pl
from jax.experimental.pallas import tpu as pltpu
from jax.experimental.pallas import tpu_sc as plsc

B = 16384
F = 32
NC = 2
NS = 16
NW = NC * NS
BPW = B // NW
L = 16
TS = 8
TL = 128
HALF = F // 2
NWAVE = BPW // L
PROWS = L * HALF


def _gmf_body(users_hbm, items_hbm, utab_hbm, itab_hbm, params_hbm, out_hbm,
              idx_u, idx_i, pan_u0, pan_u1, pan_i0, pan_i1, params_v, out_v,
              sem_u0, sem_u1, sem_i0, sem_i1):
    pan_u = (pan_u0, pan_u1)
    pan_i = (pan_i0, pan_i1)
    sem_u = (sem_u0, sem_u1)
    sem_i = (sem_i0, sem_i1)
    wid = lax.axis_index("s") * NC + lax.axis_index("c")
    base = wid * BPW

    pltpu.sync_copy(users_hbm.at[pl.ds(base, BPW)], idx_u)
    pltpu.sync_copy(items_hbm.at[pl.ds(base, BPW)], idx_i)
    pltpu.sync_copy(params_hbm, params_v)

    wv = [params_v[pl.ds(k * L, L)] for k in range(F // L)]
    bv = params_v[pl.ds(F // L * L, L)]
    w = [wv[f // L][f % L] for f in range(F)]
    bias = bv[0]
    lane = lax.iota(jnp.int32, L)

    def wave(v, carry):
        uvec = idx_u[pl.ds(v * L, L)]
        ivec = idx_i[pl.ds(v * L, L)]
        rem_u = uvec - (uvec // TL) * TL
        rem_i = ivec - (ivec // TL) * TL
        qus = [pl.multiple_of((uvec[k] // TL) * TL, TL) for k in range(L)]
        qis = [pl.multiple_of((ivec[k] // TL) * TL, TL) for k in range(L)]

        def issue(p):
            fr = p * TS
            pu, pi = pan_u[p % 2], pan_i[p % 2]
            su, si = sem_u[p % 2], sem_i[p % 2]
            cs = []
            for k in range(L):
                cs.append(pltpu.async_copy(
                    utab_hbm.at[pl.ds(fr, TS), pl.ds(qus[k], TL)],
                    pu.at[pl.ds(k * TS, TS)], su))
                cs.append(pltpu.async_copy(
                    itab_hbm.at[pl.ds(fr, TS), pl.ds(qis[k], TL)],
                    pi.at[pl.ds(k * TS, TS)], si))
            return cs

        acc = jnp.zeros((L,), jnp.float32)
        pend = issue(0)
        for p in range(F // TS):
            nxt = issue(p + 1) if p + 1 < F // TS else []
            for c in pend:
                c.wait()
            pend = nxt
            for fo in range(TS):
                f = p * TS + fo
                rows = lane * TS + fo
                ucol = plsc.load_gather(pan_u[p % 2], [rows, rem_u])
                icol = plsc.load_gather(pan_i[p % 2], [rows, rem_i])
                acc = acc + (ucol * icol) * w[f]
        z = acc + bias
        out_v[pl.ds(v * L, L)] = 1.0 / (1.0 + jnp.exp(-z))
        return carry

    lax.fori_loop(0, NWAVE, wave, 0)

    pltpu.sync_copy(out_v, out_hbm.at[pl.ds(base, BPW)])


_gmf = functools.partial(
    pl.kernel,
    out_type=jax.ShapeDtypeStruct((B,), jnp.float32),
    mesh=plsc.VectorSubcoreMesh(core_axis_name="c", subcore_axis_name="s"),
    scratch_types=[
        pltpu.VMEM((BPW,), jnp.int32),
        pltpu.VMEM((BPW,), jnp.int32),
        pltpu.VMEM((L * TS, TL), jnp.float32),
        pltpu.VMEM((L * TS, TL), jnp.float32),
        pltpu.VMEM((L * TS, TL), jnp.float32),
        pltpu.VMEM((L * TS, TL), jnp.float32),
        pltpu.VMEM((48,), jnp.float32),
        pltpu.VMEM((BPW,), jnp.float32),
        pltpu.SemaphoreType.DMA,
        pltpu.SemaphoreType.DMA,
        pltpu.SemaphoreType.DMA,
        pltpu.SemaphoreType.DMA,
    ],
    compiler_params=pltpu.CompilerParams(needs_layout_passes=False),
)(_gmf_body)


def kernel(users, items, user_table, item_table, pred_w, pred_b):
    params = jnp.concatenate([
        pred_w.reshape(-1).astype(jnp.float32),
        pred_b.reshape(-1).astype(jnp.float32),
        jnp.zeros((48 - F - 1,), jnp.float32),
    ])
    return _gmf(users.astype(jnp.int32), items.astype(jnp.int32),
                user_table.T, item_table.T, params)
